# baseline (speedup 1.0000x reference)
import dataclasses
import functools

import jax
import jax.numpy as jnp
from jax import lax
from jax.experimental import pallas as pl
from jax.experimental.pallas import tpu as pltpu
from jax.experimental.pallas import tpu_sc as plsc

N_NODES = 10000
N_EDGES = 320000
D_FEAT = 128

NUM_CORES = 2
NUM_SUBCORES = 16
NUM_WORKERS = NUM_CORES * NUM_SUBCORES
EDGES_PER_WORKER = N_EDGES // NUM_WORKERS
CHUNK = 40
NUM_CHUNKS = EDGES_PER_WORKER // CHUNK
N_PAD = 10240
ROWS_PER_TILE = N_PAD // NUM_SUBCORES
LANES = 16


def _sc_compiler_params():
  cp = pltpu.CompilerParams()
  if "needs_layout_passes" in pltpu.CompilerParams.__dataclass_fields__:
    cp = dataclasses.replace(cp, needs_layout_passes=False)
  return cp


def _sc_spmm(rows, cols, vals, b):
  mesh = plsc.VectorSubcoreMesh(core_axis_name="c", subcore_axis_name="s")

  @functools.partial(
      pl.kernel,
      compiler_params=_sc_compiler_params(),
      out_type=jax.ShapeDtypeStruct((NUM_CORES * N_PAD, D_FEAT),
                                    jnp.float32),
      mesh=mesh,
      scratch_types=[
          pltpu.VMEM((EDGES_PER_WORKER,), jnp.int32),
          pltpu.VMEM((CHUNK,), jnp.int32),
          pltpu.VMEM((CHUNK,), jnp.int32),
          pltpu.VMEM((CHUNK,), jnp.int32),
          pltpu.VMEM((CHUNK,), jnp.int32),
          pltpu.VMEM((CHUNK,), jnp.float32),
          pltpu.VMEM((CHUNK,), jnp.float32),
          pltpu.VMEM((CHUNK, D_FEAT), jnp.float32),
          pltpu.VMEM((CHUNK, D_FEAT), jnp.float32),
          pltpu.VMEM((CHUNK, D_FEAT), jnp.float32),
          pltpu.VMEM((CHUNK, D_FEAT), jnp.float32),
          pltpu.VMEM_SHARED((N_PAD, D_FEAT), jnp.float32),
          pltpu.SemaphoreType.DMA,
          pltpu.SemaphoreType.DMA,
          pltpu.SemaphoreType.DMA,
          pltpu.SemaphoreType.DMA,
      ],
  )
  def kern(rows_hbm, cols_hbm, vals_hbm, b_hbm, out_hbm,
           cols_v, r0, r1, si0, si1, v0, v1, g0, g1, s0, s1, acc,
           gsem0, gsem1, ssem0, ssem1):
    cid = lax.axis_index("c")
    sid = lax.axis_index("s")
    wid = sid * NUM_CORES + cid
    rbuf = (r0, r1)
    sibuf = (si0, si1)
    vbuf = (v0, v1)
    gbuf = (g0, g1)
    sbuf = (s0, s1)
    gsem = (gsem0, gsem1)
    ssem = (ssem0, ssem1)

    pltpu.sync_copy(cols_hbm.at[wid], cols_v)

    zeros16 = jnp.zeros((LANES,), jnp.float32)

    @pl.loop(0, CHUNK)
    def _(i):
      for k in range(D_FEAT // LANES):
        s0[i, pl.ds(k * LANES, LANES)] = zeros16

    @pl.loop(0, ROWS_PER_TILE // CHUNK)
    def _(j):
      pltpu.sync_copy(s0,
                      acc.at[pl.ds(sid * ROWS_PER_TILE + j * CHUNK, CHUNK)])

    plsc.subcore_barrier()

    def fire(c, p):
      base = wid * EDGES_PER_WORKER + c * CHUNK
      pltpu.async_copy(rows_hbm.at[pl.ds(base, CHUNK)], rbuf[p], gsem[p])
      pltpu.async_copy(vals_hbm.at[pl.ds(base, CHUNK)], vbuf[p], gsem[p])
      pltpu.async_copy(
          b_hbm.at[cols_v.at[pl.ds(c * CHUNK, CHUNK)]], gbuf[p], gsem[p])

    def wait_fire(p):
      base0 = wid * EDGES_PER_WORKER
      pltpu.make_async_copy(rows_hbm.at[pl.ds(base0, CHUNK)],
                            rbuf[p], gsem[p]).wait()
      pltpu.make_async_copy(vals_hbm.at[pl.ds(base0, CHUNK)],
                            vbuf[p], gsem[p]).wait()
      pltpu.make_async_copy(b_hbm.at[cols_v.at[pl.ds(0, CHUNK)]],
                            gbuf[p], gsem[p]).wait()

    def drain_scatter(p):
      pltpu.make_async_copy(sbuf[p], acc.at[sibuf[p]], ssem[p]).wait()

    def scale(p):
      @plsc.parallel_loop(0, CHUNK, unroll=4)
      def _(i):
        vbc = plsc.load_gather(vbuf[p], [jnp.full((LANES,), i, jnp.int32)])
        for k in range(D_FEAT // LANES):
          sl = (i, pl.ds(k * LANES, LANES))
          sbuf[p][sl] = gbuf[p][sl] * vbc

    fire(0, 0)
    fire(1, 1)

    @pl.loop(0, NUM_CHUNKS, step=2)
    def _(c0):
      for p in range(2):
        c = c0 + p
        @pl.when(c0 >= 2)
        def _():
          drain_scatter(p)
        wait_fire(p)
        scale(p)
        for off in (0, 16, 24):
          sibuf[p][pl.ds(off, LANES)] = rbuf[p][pl.ds(off, LANES)]
        pltpu.async_copy(sbuf[p], acc.at[sibuf[p]], ssem[p], add=True)
        @pl.when(c + 2 < NUM_CHUNKS)
        def _():
          fire(c + 2, p)

    for p in range(2):
      drain_scatter(p)

    plsc.subcore_barrier()

    pltpu.sync_copy(
        acc.at[pl.ds(sid * ROWS_PER_TILE, ROWS_PER_TILE)],
        out_hbm.at[pl.ds(cid * N_PAD + sid * ROWS_PER_TILE, ROWS_PER_TILE)])

  return kern(rows, cols, vals, b)


def _add_partials(p0, p1):
  def body(a_ref, b_ref, o_ref):
    o_ref[...] = a_ref[...] + b_ref[...]

  blk = 1000
  return pl.pallas_call(
      body,
      grid=(N_NODES // blk,),
      in_specs=[pl.BlockSpec((blk, D_FEAT), lambda i: (i, 0)),
                pl.BlockSpec((blk, D_FEAT), lambda i: (i, 0))],
      out_specs=pl.BlockSpec((blk, D_FEAT), lambda i: (i, 0)),
      out_shape=jax.ShapeDtypeStruct((N_NODES, D_FEAT), jnp.float32),
  )(p0, p1)


@jax.jit
def kernel(indices, values, b):
  rows = indices[0]
  cols = indices[1].reshape(NUM_WORKERS, EDGES_PER_WORKER)
  partials = _sc_spmm(rows, cols, values, b)
  return _add_partials(partials[:N_NODES], partials[N_PAD:N_PAD + N_NODES])


# speedup vs baseline: 1.0844x; 1.0844x over previous
import dataclasses
import functools

import jax
import jax.numpy as jnp
from jax import lax
from jax.experimental import pallas as pl
from jax.experimental.pallas import tpu as pltpu
from jax.experimental.pallas import tpu_sc as plsc

N_NODES = 10000
N_EDGES = 320000
D_FEAT = 128

NUM_CORES = 2
NUM_SUBCORES = 16
NUM_WORKERS = NUM_CORES * NUM_SUBCORES
EDGES_PER_WORKER = N_EDGES // NUM_WORKERS
CHUNK = 80
NUM_CHUNKS = EDGES_PER_WORKER // CHUNK
IRING = 8
N_PAD = 10240
ROWS_PER_TILE = N_PAD // NUM_SUBCORES
LANES = 16


def _sc_compiler_params():
  cp = pltpu.CompilerParams()
  if "needs_layout_passes" in pltpu.CompilerParams.__dataclass_fields__:
    cp = dataclasses.replace(cp, needs_layout_passes=False)
  return cp


def _sc_spmm(packed, b):
  mesh = plsc.VectorSubcoreMesh(core_axis_name="c", subcore_axis_name="s")

  @functools.partial(
      pl.kernel,
      compiler_params=_sc_compiler_params(),
      out_type=jax.ShapeDtypeStruct((NUM_CORES * N_PAD, D_FEAT),
                                    jnp.float32),
      mesh=mesh,
      scratch_types=(
          [pltpu.VMEM((3, CHUNK), jnp.int32) for _ in range(IRING)]
          + [pltpu.VMEM((CHUNK, D_FEAT), jnp.float32) for _ in range(4)]
          + [pltpu.VMEM_SHARED((N_PAD, D_FEAT), jnp.float32)]
          + [pltpu.SemaphoreType.DMA for _ in range(IRING + 4)]
      ),
  )
  def kern(packed_hbm, b_hbm, out_hbm, *refs):
    pbuf = refs[:IRING]
    g0, g1, s0, s1 = refs[IRING:IRING + 4]
    acc = refs[IRING + 4]
    isem = refs[IRING + 5:IRING + 5 + IRING]
    gsem = refs[IRING + 5 + IRING:IRING + 7 + IRING]
    ssem = refs[IRING + 7 + IRING:]
    gbuf = (g0, g1)
    sbuf = (s0, s1)

    cid = lax.axis_index("c")
    sid = lax.axis_index("s")
    wid = sid * NUM_CORES + cid
    cbase = wid * NUM_CHUNKS

    zeros16 = jnp.zeros((LANES,), jnp.float32)

    @pl.loop(0, CHUNK)
    def _(i):
      for k in range(D_FEAT // LANES):
        s0[i, pl.ds(k * LANES, LANES)] = zeros16

    @pl.loop(0, ROWS_PER_TILE // CHUNK)
    def _(j):
      pltpu.sync_copy(s0,
                      acc.at[pl.ds(sid * ROWS_PER_TILE + j * CHUNK, CHUNK)])

    plsc.subcore_barrier()

    def fire_idx(c, p8):
      pltpu.async_copy(packed_hbm.at[cbase + c], pbuf[p8], isem[p8])

    def wait_idx(p8):
      pltpu.make_async_copy(packed_hbm.at[cbase], pbuf[p8], isem[p8]).wait()

    def fire_gather(p8, p2):
      pltpu.async_copy(b_hbm.at[pbuf[p8].at[1]], gbuf[p2], gsem[p2])

    def wait_gather(p2):
      pltpu.make_async_copy(b_hbm.at[pbuf[0].at[1]], gbuf[p2],
                            gsem[p2]).wait()

    def fire_scatter(p8, p2):
      pltpu.async_copy(sbuf[p2], acc.at[pbuf[p8].at[0]], ssem[p2], add=True)

    def drain_scatter(p2):
      pltpu.make_async_copy(sbuf[p2], acc.at[pbuf[0].at[0]],
                            ssem[p2]).wait()

    def scale(p8, p2):
      @plsc.parallel_loop(0, CHUNK, unroll=4)
      def _(i):
        vbits = plsc.load_gather(
            pbuf[p8],
            [jnp.full((LANES,), 2, jnp.int32),
             jnp.full((LANES,), i, jnp.int32)])
        vbc = plsc.bitcast(vbits, jnp.float32)
        for k in range(D_FEAT // LANES):
          sl = (i, pl.ds(k * LANES, LANES))
          sbuf[p2][sl] = gbuf[p2][sl] * vbc

    def step(c, p8, p2, drain):
      if drain:
        drain_scatter(p2)
      if isinstance(c, int):
        if c + IRING - 2 < NUM_CHUNKS:
          fire_idx(c + IRING - 2, (c + IRING - 2) % IRING)
      else:
        @pl.when(c + IRING - 2 < NUM_CHUNKS)
        def _():
          fire_idx(c + IRING - 2, (p8 + IRING - 2) % IRING)
      wait_gather(p2)
      scale(p8, p2)
      fire_scatter(p8, p2)
      if not isinstance(c, int) or c + 2 < NUM_CHUNKS:
        wait_idx((p8 + 2) % IRING)
        fire_gather((p8 + 2) % IRING, p2)

    for j in range(IRING - 2):
      fire_idx(j, j)
    for j in range(2):
      wait_idx(j)
      fire_gather(j, j)
    for c in range(IRING):
      step(c, c % IRING, c % 2, c >= 2)

    @pl.loop(IRING, IRING * (NUM_CHUNKS // IRING), step=IRING)
    def _(c0):
      for ph in range(IRING):
        step(c0 + ph, ph, ph % 2, True)

    for c in range(IRING * (NUM_CHUNKS // IRING), NUM_CHUNKS):
      step(c, c % IRING, c % 2, True)
    for p in range(2):
      drain_scatter(p)

    plsc.subcore_barrier()

    pltpu.sync_copy(
        acc.at[pl.ds(sid * ROWS_PER_TILE, ROWS_PER_TILE)],
        out_hbm.at[pl.ds(cid * N_PAD + sid * ROWS_PER_TILE, ROWS_PER_TILE)])

  return kern(packed, b)


def _add_partials(p0, p1):
  def body(a_ref, b_ref, o_ref):
    o_ref[...] = a_ref[...] + b_ref[...]

  return pl.pallas_call(
      body,
      out_shape=jax.ShapeDtypeStruct((N_NODES, D_FEAT), jnp.float32),
  )(p0, p1)


@jax.jit
def kernel(indices, values, b):
  vbits = lax.bitcast_convert_type(values, jnp.int32)
  packed = jnp.stack(
      [indices[0].reshape(-1, CHUNK),
       indices[1].reshape(-1, CHUNK),
       vbits.reshape(-1, CHUNK)], axis=1)
  partials = _sc_spmm(packed, b)
  return _add_partials(partials[:N_NODES], partials[N_PAD:N_PAD + N_NODES])


# speedup vs baseline: 1.1025x; 1.0167x over previous
import dataclasses
import functools

import jax
import jax.numpy as jnp
from jax import lax
from jax.experimental import pallas as pl
from jax.experimental.pallas import tpu as pltpu
from jax.experimental.pallas import tpu_sc as plsc

N_NODES = 10000
N_EDGES = 320000
D_FEAT = 128

NUM_CORES = 2
NUM_SUBCORES = 16
NUM_WORKERS = NUM_CORES * NUM_SUBCORES
EDGES_PER_WORKER = N_EDGES // NUM_WORKERS
CHUNK = 80
NUM_CHUNKS = EDGES_PER_WORKER // CHUNK
IRING = 8
N_PAD = 10240
ROWS_PER_TILE = N_PAD // NUM_SUBCORES
LANES = 16


def _sc_compiler_params():
  cp = pltpu.CompilerParams()
  if "needs_layout_passes" in pltpu.CompilerParams.__dataclass_fields__:
    cp = dataclasses.replace(cp, needs_layout_passes=False)
  return cp


def _sc_spmm(packed, b):
  mesh = plsc.VectorSubcoreMesh(core_axis_name="c", subcore_axis_name="s")

  @functools.partial(
      pl.kernel,
      compiler_params=_sc_compiler_params(),
      out_type=jax.ShapeDtypeStruct((NUM_CORES * N_PAD, D_FEAT),
                                    jnp.float32),
      mesh=mesh,
      scratch_types=(
          [pltpu.VMEM((3, CHUNK), jnp.int32) for _ in range(IRING)]
          + [pltpu.VMEM((CHUNK, D_FEAT), jnp.float32) for _ in range(4)]
          + [pltpu.VMEM_SHARED((N_PAD, D_FEAT), jnp.float32)]
          + [pltpu.SemaphoreType.DMA for _ in range(IRING + 4)]
      ),
  )
  def kern(packed_hbm, b_hbm, out_hbm, *refs):
    pbuf = refs[:IRING]
    g0, g1, s0, s1 = refs[IRING:IRING + 4]
    acc = refs[IRING + 4]
    isem = refs[IRING + 5:IRING + 5 + IRING]
    gsem = refs[IRING + 5 + IRING:IRING + 7 + IRING]
    ssem = refs[IRING + 7 + IRING:]
    gbuf = (g0, g1)
    sbuf = (s0, s1)

    cid = lax.axis_index("c")
    sid = lax.axis_index("s")
    wid = sid * NUM_CORES + cid
    cbase = wid * NUM_CHUNKS

    zeros16 = jnp.zeros((LANES,), jnp.float32)

    @pl.loop(0, CHUNK)
    def _(i):
      for k in range(D_FEAT // LANES):
        s0[i, pl.ds(k * LANES, LANES)] = zeros16

    @pl.loop(0, ROWS_PER_TILE // CHUNK)
    def _(j):
      pltpu.sync_copy(s0,
                      acc.at[pl.ds(sid * ROWS_PER_TILE + j * CHUNK, CHUNK)])

    plsc.subcore_barrier()

    def fire_idx(c, p8):
      pltpu.async_copy(packed_hbm.at[cbase + c], pbuf[p8], isem[p8])

    def wait_idx(p8):
      pltpu.make_async_copy(packed_hbm.at[cbase], pbuf[p8], isem[p8]).wait()

    def fire_gather(p8, p2):
      pltpu.async_copy(b_hbm.at[pbuf[p8].at[1]], gbuf[p2], gsem[p2])

    def wait_gather(p2):
      pltpu.make_async_copy(b_hbm.at[pbuf[0].at[1]], gbuf[p2],
                            gsem[p2]).wait()

    def fire_scatter(p8, p2):
      pltpu.async_copy(sbuf[p2], acc.at[pbuf[p8].at[0]], ssem[p2], add=True)

    def drain_scatter(p2):
      pltpu.make_async_copy(sbuf[p2], acc.at[pbuf[0].at[0]],
                            ssem[p2]).wait()

    def scale(p8, p2):
      @plsc.parallel_loop(0, CHUNK, unroll=4)
      def _(i):
        vbits = plsc.load_gather(
            pbuf[p8],
            [jnp.full((LANES,), 2, jnp.int32),
             jnp.full((LANES,), i, jnp.int32)])
        vbc = plsc.bitcast(vbits, jnp.float32)
        for k in range(D_FEAT // LANES):
          sl = (i, pl.ds(k * LANES, LANES))
          sbuf[p2][sl] = gbuf[p2][sl] * vbc

    def step(c, p8, p2, drain):
      if drain:
        drain_scatter(p2)
      if isinstance(c, int):
        if c + IRING - 2 < NUM_CHUNKS:
          fire_idx(c + IRING - 2, (c + IRING - 2) % IRING)
      else:
        @pl.when(c + IRING - 2 < NUM_CHUNKS)
        def _():
          fire_idx(c + IRING - 2, (p8 + IRING - 2) % IRING)
      wait_gather(p2)
      scale(p8, p2)
      fire_scatter(p8, p2)
      if not isinstance(c, int) or c + 2 < NUM_CHUNKS:
        wait_idx((p8 + 2) % IRING)
        fire_gather((p8 + 2) % IRING, p2)

    for j in range(IRING - 2):
      fire_idx(j, j)
    for j in range(2):
      wait_idx(j)
      fire_gather(j, j)
    for c in range(IRING):
      step(c, c % IRING, c % 2, c >= 2)

    @pl.loop(IRING, IRING * (NUM_CHUNKS // IRING), step=IRING)
    def _(c0):
      for ph in range(IRING):
        step(c0 + ph, ph, ph % 2, True)

    for c in range(IRING * (NUM_CHUNKS // IRING), NUM_CHUNKS):
      step(c, c % IRING, c % 2, True)
    for p in range(2):
      drain_scatter(p)

    plsc.subcore_barrier()

    pltpu.sync_copy(
        acc.at[pl.ds(sid * ROWS_PER_TILE, ROWS_PER_TILE)],
        out_hbm.at[pl.ds(cid * N_PAD + sid * ROWS_PER_TILE, ROWS_PER_TILE)])

  return kern(packed, b)


def _add_partials(partials):
  def body(a_ref, b_ref, o_ref):
    o_ref[...] = a_ref[...] + b_ref[...]

  blk = N_PAD // 4
  return pl.pallas_call(
      body,
      grid=(4,),
      in_specs=[pl.BlockSpec((blk, D_FEAT), lambda i: (i, 0)),
                pl.BlockSpec((blk, D_FEAT), lambda i: (i + 4, 0))],
      out_specs=pl.BlockSpec((blk, D_FEAT), lambda i: (i, 0)),
      out_shape=jax.ShapeDtypeStruct((N_PAD, D_FEAT), jnp.float32),
  )(partials, partials)


@jax.jit
def kernel(indices, values, b):
  vbits = lax.bitcast_convert_type(values, jnp.int32)
  packed = jnp.stack(
      [indices[0].reshape(-1, CHUNK),
       indices[1].reshape(-1, CHUNK),
       vbits.reshape(-1, CHUNK)], axis=1)
  partials = _sc_spmm(packed, b)
  return _add_partials(partials)[:N_NODES]
